# Initial kernel scaffold; baseline (speedup 1.0000x reference)
import jax
import jax.numpy as jnp
from jax.experimental import pallas as pl


def kernel(embeddings, labels, centers):
    raise NotImplementedError("write your pallas kernel here")


# speedup vs baseline: 1.1103x; 1.1103x over previous
import functools

import jax
import jax.numpy as jnp
from jax import lax
from jax.experimental import pallas as pl
from jax.experimental.pallas import tpu as pltpu
from jax.experimental.pallas import tpu_sc as plsc

_B = 16384
_D = 512
_NUM_CLASSES = 1000
_LAMBDA = 0.001

_NC, _NS, _L = 2, 16, 16
_NW = _NC * _NS
_BPW = _B // _NW
_CH = 32
_NCHUNK = _BPW // _CH

_mesh = plsc.VectorSubcoreMesh(
    core_axis_name="c", subcore_axis_name="s",
    num_cores=_NC, num_subcores=_NS,
)


@functools.partial(
    pl.kernel,
    out_type=jax.ShapeDtypeStruct((_NW, _L), jnp.float32),
    mesh=_mesh,
    scratch_types=[
        pltpu.VMEM((_BPW,), jnp.int32),
        pltpu.VMEM((_CH, _D), jnp.float32),
        pltpu.VMEM((_CH, _D), jnp.float32),
        pltpu.VMEM((_L,), jnp.float32),
        pltpu.SemaphoreType.DMA,
    ],
)
def _center_loss_sc(emb_hbm, lab_hbm, cen_hbm, out_hbm,
                    idx_v, e_v, c_v, acc_v, sem):
    wid = lax.axis_index("s") * _NC + lax.axis_index("c")
    base = wid * _BPW
    pltpu.sync_copy(lab_hbm.at[pl.ds(base, _BPW)], idx_v)

    acc = jnp.zeros((_L,), jnp.float32)
    for ch in range(_NCHUNK):
        gather = pltpu.async_copy(
            cen_hbm.at[idx_v.at[pl.ds(ch * _CH, _CH)]], c_v, sem)
        pltpu.sync_copy(emb_hbm.at[pl.ds(base + ch * _CH, _CH), :], e_v)
        gather.wait()

        def body(r, a):
            for j in range(_D // _L):
                d = e_v[r, pl.ds(j * _L, _L)] - c_v[r, pl.ds(j * _L, _L)]
                a = a + d * d
            return a

        acc = lax.fori_loop(0, _CH, body, acc)

    acc_v[...] = acc
    pltpu.sync_copy(acc_v, out_hbm.at[wid])


def kernel(embeddings, labels, centers):
    partials = _center_loss_sc(embeddings, labels.astype(jnp.int32), centers)
    return _LAMBDA * (jnp.sum(partials) / jnp.float32(_B))


# speedup vs baseline: 1.5148x; 1.3643x over previous
import functools

import jax
import jax.numpy as jnp
from jax import lax
from jax.experimental import pallas as pl
from jax.experimental.pallas import tpu as pltpu
from jax.experimental.pallas import tpu_sc as plsc

_B = 16384
_D = 512
_NUM_CLASSES = 1000
_LAMBDA = 0.001

_NC, _NS, _L = 2, 16, 16
_NW = _NC * _NS
_BPW = _B // _NW
_CH = 32
_NCHUNK = _BPW // _CH

_mesh = plsc.VectorSubcoreMesh(
    core_axis_name="c", subcore_axis_name="s",
    num_cores=_NC, num_subcores=_NS,
)


@functools.partial(
    pl.kernel,
    out_type=jax.ShapeDtypeStruct((_NW, _L), jnp.float32),
    mesh=_mesh,
    scratch_types=[
        pltpu.VMEM((_BPW,), jnp.int32),
        pltpu.VMEM((2, _CH, _D), jnp.float32),
        pltpu.VMEM((2, _CH, _D), jnp.float32),
        pltpu.VMEM((_L,), jnp.float32),
        pltpu.SemaphoreType.DMA,
        pltpu.SemaphoreType.DMA,
    ],
)
def _center_loss_sc(emb_hbm, lab_hbm, cen_hbm, out_hbm,
                    idx_v, e_v, c_v, acc_v, sem0, sem1):
    wid = lax.axis_index("s") * _NC + lax.axis_index("c")
    base = wid * _BPW
    pltpu.sync_copy(lab_hbm.at[pl.ds(base, _BPW)], idx_v)
    sems = (sem0, sem1)

    def start(ch):
        b = ch % 2
        g = pltpu.async_copy(
            cen_hbm.at[idx_v.at[pl.ds(ch * _CH, _CH)]], c_v.at[b], sems[b])
        e = pltpu.async_copy(
            emb_hbm.at[pl.ds(base + ch * _CH, _CH), :], e_v.at[b], sems[b])
        return g, e

    inflight = start(0)
    acc = jnp.zeros((_L,), jnp.float32)
    for ch in range(_NCHUNK):
        g, e = inflight
        if ch + 1 < _NCHUNK:
            nxt = start(ch + 1)
        g.wait()
        e.wait()
        b = ch % 2

        def body(r, a):
            for j in range(_D // _L):
                d = e_v[b, r, pl.ds(j * _L, _L)] - c_v[b, r, pl.ds(j * _L, _L)]
                a = a + d * d
            return a

        acc = lax.fori_loop(0, _CH, body, acc)
        if ch + 1 < _NCHUNK:
            inflight = nxt

    acc_v[...] = acc
    pltpu.sync_copy(acc_v, out_hbm.at[wid])


def kernel(embeddings, labels, centers):
    partials = _center_loss_sc(embeddings, labels.astype(jnp.int32), centers)
    return _LAMBDA * (jnp.sum(partials) / jnp.float32(_B))


# speedup vs baseline: 1.6989x; 1.1215x over previous
import functools

import jax
import jax.numpy as jnp
from jax import lax
from jax.experimental import pallas as pl
from jax.experimental.pallas import tpu as pltpu
from jax.experimental.pallas import tpu_sc as plsc

_B = 16384
_D = 512
_NUM_CLASSES = 1000
_LAMBDA = 0.001

_NC, _NS, _L = 2, 16, 16
_NW = _NC * _NS
_BPW = _B // _NW
_CH = 32
_NCHUNK = _BPW // _CH

_mesh = plsc.VectorSubcoreMesh(
    core_axis_name="c", subcore_axis_name="s",
    num_cores=_NC, num_subcores=_NS,
)


@functools.partial(
    pl.kernel,
    out_type=jax.ShapeDtypeStruct((_NW, _L), jnp.float32),
    mesh=_mesh,
    scratch_types=[
        pltpu.VMEM((_BPW,), jnp.int32),
        pltpu.VMEM((2, _CH, _D), jnp.float32),
        pltpu.VMEM((2, _CH, _D), jnp.float32),
        pltpu.VMEM((_L,), jnp.float32),
        pltpu.SemaphoreType.DMA,
        pltpu.SemaphoreType.DMA,
    ],
)
def _center_loss_sc(emb_hbm, lab_hbm, cen_hbm, out_hbm,
                    idx_v, e_v, c_v, acc_v, sem0, sem1):
    wid = lax.axis_index("s") * _NC + lax.axis_index("c")
    base = wid * _BPW
    pltpu.sync_copy(lab_hbm.at[pl.ds(base, _BPW)], idx_v)
    sems = (sem0, sem1)

    def start(ch, b):
        pltpu.async_copy(
            cen_hbm.at[idx_v.at[pl.ds(ch * _CH, _CH)]], c_v.at[b], sems[b])
        pltpu.async_copy(
            emb_hbm.at[pl.ds(base + ch * _CH, _CH), :], e_v.at[b], sems[b])

    def wait(b):
        pltpu.make_async_copy(emb_hbm.at[pl.ds(0, _CH), :], e_v.at[b],
                              sems[b]).wait()
        pltpu.make_async_copy(emb_hbm.at[pl.ds(0, _CH), :], c_v.at[b],
                              sems[b]).wait()

    def compute(b, a):
        def body(r, a):
            for j in range(_D // _L):
                d = e_v[b, r, pl.ds(j * _L, _L)] - c_v[b, r, pl.ds(j * _L, _L)]
                a = a + d * d
            return a
        return lax.fori_loop(0, _CH, body, a)

    start(0, 0)
    npairs = _NCHUNK // 2

    def pair_body(p, a):
        start(2 * p + 1, 1)
        wait(0)
        a = compute(0, a)

        @pl.when(p + 1 < npairs)
        def _():
            start(2 * p + 2, 0)

        wait(1)
        a = compute(1, a)
        return a

    acc = lax.fori_loop(0, npairs, pair_body, jnp.zeros((_L,), jnp.float32))

    acc_v[...] = acc
    pltpu.sync_copy(acc_v, out_hbm.at[wid])


def kernel(embeddings, labels, centers):
    partials = _center_loss_sc(embeddings, labels.astype(jnp.int32), centers)
    return _LAMBDA * (jnp.sum(partials) / jnp.float32(_B))
